# baseline (speedup 1.0000x reference)
import jax
import jax.numpy as jnp
from jax.experimental import pallas as pl

TOPK = 2
E = 8


def _copy_body(x_ref, o_ref):
    o_ref[...] = x_ref[...]


def kernel(x, gate_w, w_gate, w_up, w_down):
    bsz, seq_len, hidden_dim = x.shape
    t = x.reshape(-1, hidden_dim)
    router_logits = t @ gate_w
    routing_weights = jax.nn.softmax(router_logits, axis=-1)
    top_w, top_idx = jax.lax.top_k(routing_weights, TOPK)
    top_w = top_w / jnp.sum(top_w, axis=-1, keepdims=True)
    final = jnp.zeros_like(t)
    for e in range(E):
        w_e = jnp.sum(jnp.where(top_idx == e, top_w, 0.0), axis=-1)
        h1 = jax.nn.silu(t @ w_gate[e]) * (t @ w_up[e])
        expert_out = h1 @ w_down[e]
        final = final + expert_out * w_e[:, None]
    final = pl.pallas_call(
        _copy_body,
        grid=(16,),
        in_specs=[pl.BlockSpec((256, hidden_dim), lambda i: (i, 0))],
        out_specs=pl.BlockSpec((256, hidden_dim), lambda i: (i, 0)),
        out_shape=jax.ShapeDtypeStruct(final.shape, final.dtype),
    )(final)
    return final.reshape(bsz, seq_len, hidden_dim)


# speedup vs baseline: 1.5862x; 1.5862x over previous
import functools

import jax
import jax.numpy as jnp
from jax import lax
from jax.experimental import pallas as pl
from jax.experimental.pallas import tpu as pltpu
from jax.experimental.pallas import tpu_sc as plsc

T = 4096
H = 2048
FF = 1408
E = 8
A = 2 * T
BLK = 256
NT = A // BLK + E
P = NT * BLK
NW = 32
CH = 16
NCH = (A // NW) // CH

_ABLK = 512
_NABLK = A // _ABLK


def _router_body(x_ref, gw_ref, pos_ref, w1_ref, w2_ref, eot_ref, logits):
    i = pl.program_id(0)
    nx = pl.num_programs(0) - 1

    @pl.when(i < nx)
    def _():
        xb = x_ref[...].astype(jnp.bfloat16)
        gw = gw_ref[...].astype(jnp.bfloat16)
        logits[pl.ds(i * BLK, BLK), :] = lax.dot(
            xb, gw, preferred_element_type=jnp.float32)

    @pl.when(i == nx)
    def _():
        lg = logits[...]
        lane = lax.broadcasted_iota(jnp.int32, (T, E), 1)
        m1 = jnp.max(lg, axis=-1, keepdims=True)
        i1 = jnp.min(jnp.where(lg == m1, lane, E), axis=-1, keepdims=True)
        lg2 = jnp.where(lane == i1, -jnp.inf, lg)
        m2 = jnp.max(lg2, axis=-1, keepdims=True)
        i2 = jnp.min(jnp.where(lg2 == m2, lane, E), axis=-1, keepdims=True)
        s2 = jnp.exp(m2 - m1)
        w1 = 1.0 / (1.0 + s2)
        w2 = s2 / (1.0 + s2)
        w1_ref[...] = w1
        w2_ref[...] = w2

        e_asn = jnp.concatenate([i1, i2], axis=0)
        lane_e = lax.broadcasted_iota(jnp.int32, (A, E), 1)
        onehot = (e_asn == lane_e).astype(jnp.float32)

        r = lax.broadcasted_iota(jnp.int32, (_ABLK, _ABLK), 0)
        c = lax.broadcasted_iota(jnp.int32, (_ABLK, _ABLK), 1)
        tril = (c < r).astype(jnp.float32)
        ranks = []
        tots = []
        for b in range(_NABLK):
            ob = onehot[b * _ABLK:(b + 1) * _ABLK, :]
            ranks.append(lax.dot(tril, ob, preferred_element_type=jnp.float32))
            tots.append(jnp.sum(ob, axis=0, keepdims=True))
        tot = jnp.concatenate(tots, axis=0)
        rb = lax.broadcasted_iota(jnp.int32, (_NABLK, _NABLK), 0)
        cb = lax.broadcasted_iota(jnp.int32, (_NABLK, _NABLK), 1)
        trilb = (cb < rb).astype(jnp.float32)
        off = lax.dot(trilb, tot, preferred_element_type=jnp.float32)
        rank = jnp.concatenate(
            [ranks[b] + off[b:b + 1, :] for b in range(_NABLK)], axis=0)

        counts = jnp.sum(tot, axis=0, keepdims=True)
        seg = jnp.floor((counts + (BLK - 1)) * (1.0 / BLK)) * BLK
        re8 = lax.broadcasted_iota(jnp.int32, (E, E), 0)
        ce8 = lax.broadcasted_iota(jnp.int32, (E, E), 1)
        tril8 = (re8 < ce8).astype(jnp.float32)
        start = lax.dot(seg, tril8, preferred_element_type=jnp.float32)

        pos = jnp.sum((rank + start) * onehot, axis=-1, keepdims=True)
        pos_ref[...] = pos.astype(jnp.int32)

        ptile = lax.broadcasted_iota(jnp.int32, (128, E), 0).astype(jnp.float32) * BLK
        cmp = (jnp.broadcast_to(start, (128, E)) <= ptile).astype(jnp.float32)
        eot = jnp.sum(cmp, axis=-1, keepdims=True) - 1.0
        eot_ref[...] = eot.astype(jnp.int32)


def _router(x2d, gate_w):
    nx = T // BLK
    return pl.pallas_call(
        _router_body,
        grid=(nx + 1,),
        in_specs=[
            pl.BlockSpec((BLK, H), lambda i: (jnp.minimum(i, 15), 0)),
            pl.BlockSpec((H, E), lambda i: (0, 0)),
        ],
        out_specs=[
            pl.BlockSpec((A, 1), lambda i: (0, 0)),
            pl.BlockSpec((T, 1), lambda i: (0, 0)),
            pl.BlockSpec((T, 1), lambda i: (0, 0)),
            pl.BlockSpec((128, 1), lambda i: (0, 0)),
        ],
        out_shape=[
            jax.ShapeDtypeStruct((A, 1), jnp.int32),
            jax.ShapeDtypeStruct((T, 1), jnp.float32),
            jax.ShapeDtypeStruct((T, 1), jnp.float32),
            jax.ShapeDtypeStruct((128, 1), jnp.int32),
        ],
        scratch_shapes=[pltpu.VMEM((T, E), jnp.float32)],
    )(x2d, gate_w)


def _sc_dispatch_body(x_hbm, posr_hbm, xs_hbm, idx_v, rows_a, rows_b, sin, souta, soutb):
    w = lax.axis_index("s") * 2 + lax.axis_index("c")
    tbase = (w % 16) * 256
    pltpu.sync_copy(posr_hbm.at[w], idx_v)
    bufs = (rows_a, rows_b)
    outs = (souta, soutb)
    pltpu.make_async_copy(
        x_hbm.at[pl.ds(tbase, CH)], rows_a, sin).start()
    for c in range(NCH):
        cur = bufs[c % 2]
        nxt = bufs[(c + 1) % 2]
        pltpu.make_async_copy(
            x_hbm.at[pl.ds(tbase + c * CH, CH)], cur, sin).wait()
        if c + 1 < NCH:
            if c >= 1:
                pltpu.make_async_copy(
                    nxt, xs_hbm.at[idx_v.at[c - 1]], outs[(c - 1) % 2]).wait()
            pltpu.make_async_copy(
                x_hbm.at[pl.ds(tbase + (c + 1) * CH, CH)], nxt, sin).start()
        pltpu.make_async_copy(cur, xs_hbm.at[idx_v.at[c]], outs[c % 2]).start()
    pltpu.make_async_copy(
        bufs[(NCH - 2) % 2], xs_hbm.at[idx_v.at[NCH - 2]], outs[(NCH - 2) % 2]).wait()
    pltpu.make_async_copy(
        bufs[(NCH - 1) % 2], xs_hbm.at[idx_v.at[NCH - 1]], outs[(NCH - 1) % 2]).wait()


def _gmm_body(eot_ref, xs_ref, wg_ref, wu_ref, wd_ref, y_ref):
    xb = xs_ref[...].astype(jnp.bfloat16)
    a = lax.dot(xb, wg_ref[0], preferred_element_type=jnp.float32)
    b = lax.dot(xb, wu_ref[0], preferred_element_type=jnp.float32)
    h = (a * jax.nn.sigmoid(a) * b).astype(jnp.bfloat16)
    y_ref[...] = lax.dot(h, wd_ref[0], preferred_element_type=jnp.float32)


def _gmm(xs, wg_bf, wu_bf, wd_bf, eot):
    grid_spec = pltpu.PrefetchScalarGridSpec(
        num_scalar_prefetch=1,
        grid=(NT,),
        in_specs=[
            pl.BlockSpec((BLK, H), lambda i, eot_ref: (i, 0)),
            pl.BlockSpec((1, H, FF), lambda i, eot_ref: (eot_ref[i], 0, 0)),
            pl.BlockSpec((1, H, FF), lambda i, eot_ref: (eot_ref[i], 0, 0)),
            pl.BlockSpec((1, FF, H), lambda i, eot_ref: (eot_ref[i], 0, 0)),
        ],
        out_specs=pl.BlockSpec((BLK, H), lambda i, eot_ref: (i, 0)),
    )
    return pl.pallas_call(
        _gmm_body,
        grid_spec=grid_spec,
        out_shape=jax.ShapeDtypeStruct((P, H), jnp.float32),
    )(eot, xs, wg_bf, wu_bf, wd_bf)


_TPW = T // NW
_CCH = 8
_NCC = _TPW // _CCH


def _sc_combine_body(y_hbm, posc_hbm, wc_hbm, out_hbm, idx_v, wv, rows, orows, sem):
    w = lax.axis_index("s") * 2 + lax.axis_index("c")
    pltpu.sync_copy(posc_hbm.at[pl.ds(w * _NCC, _NCC)], idx_v)
    pltpu.sync_copy(wc_hbm.at[pl.ds(w * _NCC, _NCC)], wv)
    for c in range(_NCC):
        pltpu.make_async_copy(y_hbm.at[idx_v.at[c]], rows, sem).start()
        pltpu.make_async_copy(y_hbm.at[idx_v.at[c]], rows, sem).wait()
        wrow = wv[c]
        for j in range(_CCH):
            w1 = wrow[j]
            w2 = wrow[j + _CCH]

            def body(i, _, j=j, w1=w1, w2=w2):
                r0 = rows[j, pl.ds(i * 16, 16)]
                r1 = rows[j + _CCH, pl.ds(i * 16, 16)]
                orows[j, pl.ds(i * 16, 16)] = w1 * r0 + w2 * r1
                return ()

            lax.fori_loop(0, H // 16, body, (), unroll=4)
        pltpu.sync_copy(orows, out_hbm.at[pl.ds(w * _TPW + c * _CCH, _CCH)])


@functools.cache
def _sc_kernels():
    mesh = plsc.VectorSubcoreMesh(
        core_axis_name="c", subcore_axis_name="s", num_cores=2, num_subcores=16)
    dispatch = pl.kernel(
        _sc_dispatch_body,
        out_type=jax.ShapeDtypeStruct((P, H), jnp.float32),
        mesh=mesh,
        scratch_types=[
            pltpu.VMEM((NCH, CH), jnp.int32),
            pltpu.VMEM((CH, H), jnp.float32),
            pltpu.VMEM((CH, H), jnp.float32),
            pltpu.SemaphoreType.DMA,
            pltpu.SemaphoreType.DMA,
            pltpu.SemaphoreType.DMA,
        ],
    )
    combine = pl.kernel(
        _sc_combine_body,
        out_type=jax.ShapeDtypeStruct((T, H), jnp.float32),
        mesh=mesh,
        scratch_types=[
            pltpu.VMEM((_NCC, 2 * _CCH), jnp.int32),
            pltpu.VMEM((_NCC, 2 * _CCH), jnp.float32),
            pltpu.VMEM((2 * _CCH, H), jnp.float32),
            pltpu.VMEM((_CCH, H), jnp.float32),
            pltpu.SemaphoreType.DMA,
        ],
    )
    return dispatch, combine


def kernel(x, gate_w, w_gate, w_up, w_down):
    bsz, seq_len, hidden = x.shape
    x2d = x.reshape(T, H)
    pos, w1, w2, eot128 = _router(x2d, gate_w)

    pos_flat = pos.reshape(A)
    posr = pos_flat.reshape(NW, NCH, CH)
    p0 = pos_flat[:T].reshape(T // _CCH, _CCH)
    p1 = pos_flat[T:].reshape(T // _CCH, _CCH)
    posc = jnp.concatenate([p0, p1], axis=1)
    wc = jnp.concatenate([w1.reshape(T // _CCH, _CCH),
                          w2.reshape(T // _CCH, _CCH)], axis=1)
    eot = eot128.reshape(128)[:NT]

    dispatch, combine = _sc_kernels()
    xs = dispatch(x2d, posr)
    wg_bf = w_gate.astype(jnp.bfloat16)
    wu_bf = w_up.astype(jnp.bfloat16)
    wd_bf = w_down.astype(jnp.bfloat16)
    y = _gmm(xs, wg_bf, wu_bf, wd_bf, eot)
    out = combine(y, posc, wc)
    return out.reshape(bsz, seq_len, hidden)


# speedup vs baseline: 1.8243x; 1.1501x over previous
import functools

import jax
import jax.numpy as jnp
from jax import lax
from jax.experimental import pallas as pl
from jax.experimental.pallas import tpu as pltpu
from jax.experimental.pallas import tpu_sc as plsc

T = 4096
H = 2048
FF = 1408
E = 8
A = 2 * T
BLK = 256
NT = A // BLK + E
P = NT * BLK
NW = 32
CH = 16
NCH = (A // NW) // CH

_ABLK = 512
_NABLK = A // _ABLK


def _router_body(x_ref, gw_ref, pos_ref, w1_ref, w2_ref, eot_ref, logits):
    i = pl.program_id(0)
    nx = pl.num_programs(0) - 1

    @pl.when(i < nx)
    def _():
        xb = x_ref[...].astype(jnp.bfloat16)
        gw = gw_ref[...].astype(jnp.bfloat16)
        logits[pl.ds(i * BLK, BLK), :] = lax.dot(
            xb, gw, preferred_element_type=jnp.float32)

    @pl.when(i == nx)
    def _():
        lg = logits[...]
        lane = lax.broadcasted_iota(jnp.int32, (T, E), 1)
        m1 = jnp.max(lg, axis=-1, keepdims=True)
        i1 = jnp.min(jnp.where(lg == m1, lane, E), axis=-1, keepdims=True)
        lg2 = jnp.where(lane == i1, -jnp.inf, lg)
        m2 = jnp.max(lg2, axis=-1, keepdims=True)
        i2 = jnp.min(jnp.where(lg2 == m2, lane, E), axis=-1, keepdims=True)
        s2 = jnp.exp(m2 - m1)
        w1 = 1.0 / (1.0 + s2)
        w2 = s2 / (1.0 + s2)
        w1_ref[...] = w1
        w2_ref[...] = w2

        e_asn = jnp.concatenate([i1, i2], axis=0)
        lane_e = lax.broadcasted_iota(jnp.int32, (A, E), 1)
        onehot = (e_asn == lane_e).astype(jnp.float32)

        r = lax.broadcasted_iota(jnp.int32, (_ABLK, _ABLK), 0)
        c = lax.broadcasted_iota(jnp.int32, (_ABLK, _ABLK), 1)
        tril = (c < r).astype(jnp.float32)
        ranks = []
        tots = []
        for b in range(_NABLK):
            ob = onehot[b * _ABLK:(b + 1) * _ABLK, :]
            ranks.append(lax.dot(tril, ob, preferred_element_type=jnp.float32))
            tots.append(jnp.sum(ob, axis=0, keepdims=True))
        tot = jnp.concatenate(tots, axis=0)
        rb = lax.broadcasted_iota(jnp.int32, (_NABLK, _NABLK), 0)
        cb = lax.broadcasted_iota(jnp.int32, (_NABLK, _NABLK), 1)
        trilb = (cb < rb).astype(jnp.float32)
        off = lax.dot(trilb, tot, preferred_element_type=jnp.float32)
        rank = jnp.concatenate(
            [ranks[b] + off[b:b + 1, :] for b in range(_NABLK)], axis=0)

        counts = jnp.sum(tot, axis=0, keepdims=True)
        seg = jnp.floor((counts + (BLK - 1)) * (1.0 / BLK)) * BLK
        re8 = lax.broadcasted_iota(jnp.int32, (E, E), 0)
        ce8 = lax.broadcasted_iota(jnp.int32, (E, E), 1)
        tril8 = (re8 < ce8).astype(jnp.float32)
        start = lax.dot(seg, tril8, preferred_element_type=jnp.float32)

        pos = jnp.sum((rank + start) * onehot, axis=-1, keepdims=True)
        pos_ref[...] = pos.astype(jnp.int32)

        ptile = lax.broadcasted_iota(jnp.int32, (128, E), 0).astype(jnp.float32) * BLK
        cmp = (jnp.broadcast_to(start, (128, E)) <= ptile).astype(jnp.float32)
        eot = jnp.sum(cmp, axis=-1, keepdims=True) - 1.0
        eot_ref[...] = eot.astype(jnp.int32)


def _router(x2d, gate_w):
    nx = T // BLK
    return pl.pallas_call(
        _router_body,
        grid=(nx + 1,),
        in_specs=[
            pl.BlockSpec((BLK, H), lambda i: (jnp.minimum(i, 15), 0)),
            pl.BlockSpec((H, E), lambda i: (0, 0)),
        ],
        out_specs=[
            pl.BlockSpec((A, 1), lambda i: (0, 0)),
            pl.BlockSpec((T, 1), lambda i: (0, 0)),
            pl.BlockSpec((T, 1), lambda i: (0, 0)),
            pl.BlockSpec((128, 1), lambda i: (0, 0)),
        ],
        out_shape=[
            jax.ShapeDtypeStruct((A, 1), jnp.int32),
            jax.ShapeDtypeStruct((T, 1), jnp.float32),
            jax.ShapeDtypeStruct((T, 1), jnp.float32),
            jax.ShapeDtypeStruct((128, 1), jnp.int32),
        ],
        scratch_shapes=[pltpu.VMEM((T, E), jnp.float32)],
    )(x2d, gate_w)


def _sc_dispatch_body(x_hbm, posr_hbm, xs_hbm, idx_v, rows_a, rows_b, sin, souta, soutb):
    w = lax.axis_index("s") * 2 + lax.axis_index("c")
    tbase = (w % 16) * 256
    pltpu.sync_copy(posr_hbm.at[w], idx_v)
    bufs = (rows_a, rows_b)
    outs = (souta, soutb)
    pltpu.make_async_copy(
        x_hbm.at[pl.ds(tbase, CH)], rows_a, sin).start()
    for c in range(NCH):
        cur = bufs[c % 2]
        nxt = bufs[(c + 1) % 2]
        pltpu.make_async_copy(
            x_hbm.at[pl.ds(tbase + c * CH, CH)], cur, sin).wait()
        if c + 1 < NCH:
            if c >= 1:
                pltpu.make_async_copy(
                    nxt, xs_hbm.at[idx_v.at[c - 1]], outs[(c - 1) % 2]).wait()
            pltpu.make_async_copy(
                x_hbm.at[pl.ds(tbase + (c + 1) * CH, CH)], nxt, sin).start()
        pltpu.make_async_copy(cur, xs_hbm.at[idx_v.at[c]], outs[c % 2]).start()
    pltpu.make_async_copy(
        bufs[(NCH - 2) % 2], xs_hbm.at[idx_v.at[NCH - 2]], outs[(NCH - 2) % 2]).wait()
    pltpu.make_async_copy(
        bufs[(NCH - 1) % 2], xs_hbm.at[idx_v.at[NCH - 1]], outs[(NCH - 1) % 2]).wait()


def _gmm_body(eot_ref, xs_ref, wg_ref, wu_ref, wd_ref, y_ref):
    xb = xs_ref[...].astype(jnp.bfloat16)
    a = lax.dot(xb, wg_ref[0], preferred_element_type=jnp.float32)
    b = lax.dot(xb, wu_ref[0], preferred_element_type=jnp.float32)
    h = (a * jax.nn.sigmoid(a) * b).astype(jnp.bfloat16)
    y_ref[...] = lax.dot(h, wd_ref[0], preferred_element_type=jnp.float32)


def _gmm(xs, wg_bf, wu_bf, wd_bf, eot):
    grid_spec = pltpu.PrefetchScalarGridSpec(
        num_scalar_prefetch=1,
        grid=(NT,),
        in_specs=[
            pl.BlockSpec((BLK, H), lambda i, eot_ref: (i, 0)),
            pl.BlockSpec((1, H, FF), lambda i, eot_ref: (eot_ref[i], 0, 0)),
            pl.BlockSpec((1, H, FF), lambda i, eot_ref: (eot_ref[i], 0, 0)),
            pl.BlockSpec((1, FF, H), lambda i, eot_ref: (eot_ref[i], 0, 0)),
        ],
        out_specs=pl.BlockSpec((BLK, H), lambda i, eot_ref: (i, 0)),
    )
    return pl.pallas_call(
        _gmm_body,
        grid_spec=grid_spec,
        out_shape=jax.ShapeDtypeStruct((P, H), jnp.float32),
    )(eot, xs, wg_bf, wu_bf, wd_bf)


_CH2 = 16
_NC2 = (A // NW) // _CH2


def _sc_gather_body(y_hbm, posr2_hbm, g_hbm, idx_v, rows_a, rows_b,
                    sin_a, sin_b, souta, soutb):
    w = lax.axis_index("s") * 2 + lax.axis_index("c")
    abase = w * (A // NW)
    pltpu.sync_copy(posr2_hbm.at[w], idx_v)
    bufs = (rows_a, rows_b)
    sins = (sin_a, sin_b)
    outs = (souta, soutb)
    pltpu.make_async_copy(y_hbm.at[idx_v.at[0]], rows_a, sin_a).start()
    for c in range(_NC2):
        cur = bufs[c % 2]
        pltpu.make_async_copy(y_hbm.at[idx_v.at[c]], cur, sins[c % 2]).wait()
        if c + 1 < _NC2:
            nxt = bufs[(c + 1) % 2]
            if c >= 1:
                pltpu.make_async_copy(
                    nxt, g_hbm.at[pl.ds(abase + (c - 1) * _CH2, _CH2)],
                    outs[(c - 1) % 2]).wait()
            pltpu.make_async_copy(
                y_hbm.at[idx_v.at[c + 1]], nxt, sins[(c + 1) % 2]).start()
        pltpu.make_async_copy(
            cur, g_hbm.at[pl.ds(abase + c * _CH2, _CH2)], outs[c % 2]).start()
    pltpu.make_async_copy(
        bufs[(_NC2 - 2) % 2], g_hbm.at[pl.ds(abase + (_NC2 - 2) * _CH2, _CH2)],
        outs[(_NC2 - 2) % 2]).wait()
    pltpu.make_async_copy(
        bufs[(_NC2 - 1) % 2], g_hbm.at[pl.ds(abase + (_NC2 - 1) * _CH2, _CH2)],
        outs[(_NC2 - 1) % 2]).wait()


def _combine_body(g0_ref, g1_ref, w1_ref, w2_ref, o_ref):
    o_ref[...] = w1_ref[...] * g0_ref[...] + w2_ref[...] * g1_ref[...]


def _combine(g, w1, w2):
    return pl.pallas_call(
        _combine_body,
        grid=(T // BLK,),
        in_specs=[
            pl.BlockSpec((BLK, H), lambda i: (i, 0)),
            pl.BlockSpec((BLK, H), lambda i: (i + T // BLK, 0)),
            pl.BlockSpec((BLK, 1), lambda i: (i, 0)),
            pl.BlockSpec((BLK, 1), lambda i: (i, 0)),
        ],
        out_specs=pl.BlockSpec((BLK, H), lambda i: (i, 0)),
        out_shape=jax.ShapeDtypeStruct((T, H), jnp.float32),
    )(g, g, w1, w2)


@functools.cache
def _sc_kernels():
    mesh = plsc.VectorSubcoreMesh(
        core_axis_name="c", subcore_axis_name="s", num_cores=2, num_subcores=16)
    dispatch = pl.kernel(
        _sc_dispatch_body,
        out_type=jax.ShapeDtypeStruct((P, H), jnp.float32),
        mesh=mesh,
        scratch_types=[
            pltpu.VMEM((NCH, CH), jnp.int32),
            pltpu.VMEM((CH, H), jnp.float32),
            pltpu.VMEM((CH, H), jnp.float32),
            pltpu.SemaphoreType.DMA,
            pltpu.SemaphoreType.DMA,
            pltpu.SemaphoreType.DMA,
        ],
    )
    gather = pl.kernel(
        _sc_gather_body,
        out_type=jax.ShapeDtypeStruct((A, H), jnp.float32),
        mesh=mesh,
        scratch_types=[
            pltpu.VMEM((_NC2, _CH2), jnp.int32),
            pltpu.VMEM((_CH2, H), jnp.float32),
            pltpu.VMEM((_CH2, H), jnp.float32),
            pltpu.SemaphoreType.DMA,
            pltpu.SemaphoreType.DMA,
            pltpu.SemaphoreType.DMA,
            pltpu.SemaphoreType.DMA,
        ],
    )
    return dispatch, gather


def kernel(x, gate_w, w_gate, w_up, w_down):
    bsz, seq_len, hidden = x.shape
    x2d = x.reshape(T, H)
    pos, w1, w2, eot128 = _router(x2d, gate_w)

    pos_flat = pos.reshape(A)
    posr = pos_flat.reshape(NW, NCH, CH)
    posr2 = pos_flat.reshape(NW, _NC2, _CH2)
    eot = eot128.reshape(128)[:NT]

    dispatch, gather = _sc_kernels()
    xs = dispatch(x2d, posr)
    wg_bf = w_gate.astype(jnp.bfloat16)
    wu_bf = w_up.astype(jnp.bfloat16)
    wd_bf = w_down.astype(jnp.bfloat16)
    y = _gmm(xs, wg_bf, wu_bf, wd_bf, eot)
    g = gather(y, posr2)
    out = _combine(g, w1, w2)
    return out.reshape(bsz, seq_len, hidden)
